# Initial kernel scaffold; baseline (speedup 1.0000x reference)
import jax
import jax.numpy as jnp
from jax.experimental import pallas as pl


def kernel(x, edge_index, batch, Wc, bc, gamma, beta, Wf, bf):
    raise NotImplementedError("write your pallas kernel here")


# speedup vs baseline: 4.6065x; 4.6065x over previous
import functools

import jax
import jax.numpy as jnp
from jax import lax
from jax.experimental import pallas as pl
from jax.experimental.pallas import tpu as pltpu
from jax.experimental.pallas import tpu_sc as plsc

N = 10000
NP = 10240
E = 160000
EP = 163840
D = 256
HD = 128
O = 10
G = 128
L = 4
EPS = 1e-5
RB = 512
NRB = NP // RB
RPT = NP // 16
EC = 128
_PREC = lax.Precision.HIGHEST

def _sc_scatter_body(hn_lo, hn_hi, src2d, dst2d, zeros_hbm, out_lo, out_hi,
                     src_v, dst_v, rows_v, acc, sem):
    cid = lax.axis_index("c")
    sid = lax.axis_index("s")
    sl = pl.ds(sid * RPT, RPT)
    pltpu.sync_copy(zeros_hbm, acc.at[sl])
    pltpu.sync_copy(src2d.at[pl.ds(sid * 80, 80)], src_v)
    pltpu.sync_copy(dst2d.at[pl.ds(sid * 80, 80)], dst_v)
    plsc.subcore_barrier()

    def run(hn_ref):
        @pl.loop(0, 80)
        def _(j):
            pltpu.async_copy(hn_ref.at[src_v.at[j]], rows_v, sem).wait()
            pltpu.sync_copy(rows_v, acc.at[dst_v.at[j]], add=True)

    @pl.when(cid == 0)
    def _():
        run(hn_lo)

    @pl.when(cid == 1)
    def _():
        run(hn_hi)

    plsc.subcore_barrier()

    @pl.when(cid == 0)
    def _():
        pltpu.sync_copy(acc.at[sl], out_lo.at[sl])

    @pl.when(cid == 1)
    def _():
        pltpu.sync_copy(acc.at[sl], out_hi.at[sl])


@functools.cache
def _sc_kernels():
    mesh = plsc.VectorSubcoreMesh(core_axis_name="c", subcore_axis_name="s")
    sc_scatter = pl.kernel(
        _sc_scatter_body,
        out_type=(jax.ShapeDtypeStruct((NP, HD), jnp.float32),
                  jax.ShapeDtypeStruct((NP, HD), jnp.float32)),
        mesh=mesh,
        scratch_types=[
            pltpu.VMEM((80, EC), jnp.int32),
            pltpu.VMEM((80, EC), jnp.int32),
            pltpu.VMEM((EC, HD), jnp.float32),
            pltpu.VMEM_SHARED((NP, HD), jnp.float32),
            pltpu.SemaphoreType.DMA,
        ],
    )
    return sc_scatter


def _dinv_body(deg_ref, dinv_ref):
    deg = jnp.sum(deg_ref[...], axis=1) * (1.0 / HD)
    dinv_ref[...] = lax.rsqrt(1.0 + deg)


_t_dinv = pl.pallas_call(
    _dinv_body,
    out_shape=jax.ShapeDtypeStruct((NP,), jnp.float32),
)


def _ta_body(x_ref, w_ref, wf_ref, dinv_ref, hnlo_ref, hnhi_ref, z_ref):
    xb = x_ref[...]
    hn = jnp.dot(xb, w_ref[...], precision=_PREC) * dinv_ref[...][:, None]
    hnlo_ref[...] = hn[:, :HD]
    hnhi_ref[...] = hn[:, HD:]
    z_ref[...] = jnp.dot(xb, wf_ref[...], precision=_PREC)


_t_a = pl.pallas_call(
    _ta_body,
    grid=(NRB,),
    in_specs=[
        pl.BlockSpec((RB, D), lambda r: (r, 0)),
        pl.BlockSpec((D, D), lambda r: (0, 0)),
        pl.BlockSpec((D, O), lambda r: (0, 0)),
        pl.BlockSpec((RB,), lambda r: (r,)),
    ],
    out_specs=[
        pl.BlockSpec((RB, HD), lambda r: (r, 0)),
        pl.BlockSpec((RB, HD), lambda r: (r, 0)),
        pl.BlockSpec((RB, O), lambda r: (r, 0)),
    ],
    out_shape=[
        jax.ShapeDtypeStruct((NP, HD), jnp.float32),
        jax.ShapeDtypeStruct((NP, HD), jnp.float32),
        jax.ShapeDtypeStruct((NP, O), jnp.float32),
    ],
)


def _tb_body(acclo_ref, acchi_ref, hnlo_ref, hnhi_ref, dinv_ref, b_ref,
             y_ref, sums_ref):
    r = pl.program_id(0)
    a = jnp.concatenate(
        [acclo_ref[...] + hnlo_ref[...], acchi_ref[...] + hnhi_ref[...]], axis=1)
    y = a * dinv_ref[...][:, None] + b_ref[...][None, :]
    rowid = lax.broadcasted_iota(jnp.int32, (RB, 1), 0) + r * RB
    y = jnp.where(rowid < N, y, 0.0)
    y_ref[...] = y

    @pl.when(r == 0)
    def _():
        sums_ref[...] = jnp.zeros_like(sums_ref)

    sums_ref[...] += jnp.stack([jnp.sum(y, axis=0), jnp.sum(y * y, axis=0)])


_t_b = pl.pallas_call(
    _tb_body,
    grid=(NRB,),
    in_specs=[
        pl.BlockSpec((RB, HD), lambda r: (r, 0)),
        pl.BlockSpec((RB, HD), lambda r: (r, 0)),
        pl.BlockSpec((RB, HD), lambda r: (r, 0)),
        pl.BlockSpec((RB, HD), lambda r: (r, 0)),
        pl.BlockSpec((RB,), lambda r: (r,)),
        pl.BlockSpec((D,), lambda r: (0,)),
    ],
    out_specs=[
        pl.BlockSpec((RB, D), lambda r: (r, 0)),
        pl.BlockSpec((2, D), lambda r: (0, 0)),
    ],
    out_shape=[
        jax.ShapeDtypeStruct((NP, D), jnp.float32),
        jax.ShapeDtypeStruct((2, D), jnp.float32),
    ],
)


def _norm_relu(y, sums, g, be, r):
    m = sums[0] * (1.0 / N)
    var = sums[1] * (1.0 / N) - m * m
    rstd = lax.rsqrt(var + EPS)
    scale = g * rstd
    shift = be - m * scale
    h = jnp.maximum(y * scale[None, :] + shift[None, :], 0.0)
    rowid = lax.broadcasted_iota(jnp.int32, (RB, 1), 0) + r * RB
    return jnp.where(rowid < N, h, 0.0)


def _tcz_body(y_ref, sums_ref, g_ref, be_ref, w_ref, wf_ref, zin_ref, dinv_ref,
              hnlo_ref, hnhi_ref, z_ref):
    r = pl.program_id(0)
    h = _norm_relu(y_ref[...], sums_ref[...], g_ref[...], be_ref[...], r)
    hn = jnp.dot(h, w_ref[...], precision=_PREC) * dinv_ref[...][:, None]
    hnlo_ref[...] = hn[:, :HD]
    hnhi_ref[...] = hn[:, HD:]
    z_ref[...] = zin_ref[...] + jnp.dot(h, wf_ref[...], precision=_PREC)


_t_cz = pl.pallas_call(
    _tcz_body,
    grid=(NRB,),
    in_specs=[
        pl.BlockSpec((RB, D), lambda r: (r, 0)),
        pl.BlockSpec((2, D), lambda r: (0, 0)),
        pl.BlockSpec((D,), lambda r: (0,)),
        pl.BlockSpec((D,), lambda r: (0,)),
        pl.BlockSpec((D, D), lambda r: (0, 0)),
        pl.BlockSpec((D, O), lambda r: (0, 0)),
        pl.BlockSpec((RB, O), lambda r: (r, 0)),
        pl.BlockSpec((RB,), lambda r: (r,)),
    ],
    out_specs=[
        pl.BlockSpec((RB, HD), lambda r: (r, 0)),
        pl.BlockSpec((RB, HD), lambda r: (r, 0)),
        pl.BlockSpec((RB, O), lambda r: (r, 0)),
    ],
    out_shape=[
        jax.ShapeDtypeStruct((NP, HD), jnp.float32),
        jax.ShapeDtypeStruct((NP, HD), jnp.float32),
        jax.ShapeDtypeStruct((NP, O), jnp.float32),
    ],
)


def _tcf_body(y_ref, sums_ref, g_ref, be_ref, wf_ref, zin_ref, z_ref):
    r = pl.program_id(0)
    h = _norm_relu(y_ref[...], sums_ref[...], g_ref[...], be_ref[...], r)
    z_ref[...] = zin_ref[...] + jnp.dot(h, wf_ref[...], precision=_PREC)


_t_cf = pl.pallas_call(
    _tcf_body,
    grid=(NRB,),
    in_specs=[
        pl.BlockSpec((RB, D), lambda r: (r, 0)),
        pl.BlockSpec((2, D), lambda r: (0, 0)),
        pl.BlockSpec((D,), lambda r: (0,)),
        pl.BlockSpec((D,), lambda r: (0,)),
        pl.BlockSpec((D, O), lambda r: (0, 0)),
        pl.BlockSpec((RB, O), lambda r: (r, 0)),
    ],
    out_specs=pl.BlockSpec((RB, O), lambda r: (r, 0)),
    out_shape=jax.ShapeDtypeStruct((NP, O), jnp.float32),
)


def _tf_body(z_ref, batch_ref, bf_ref, out_ref):
    r = pl.program_id(0)
    gids = lax.broadcasted_iota(jnp.int32, (G, RB), 0)
    oh = (gids == batch_ref[...][None, :]).astype(jnp.float32)
    part = jnp.dot(oh, z_ref[...], precision=_PREC)

    @pl.when(r == 0)
    def _():
        out_ref[...] = jnp.zeros_like(out_ref)

    out_ref[...] += part

    @pl.when(r == NRB - 1)
    def _():
        t = out_ref[...] + jnp.sum(bf_ref[...], axis=0)[None, :]
        mx = jnp.max(t, axis=1, keepdims=True)
        lse = jnp.log(jnp.sum(jnp.exp(t - mx), axis=1, keepdims=True))
        out_ref[...] = t - mx - lse


_t_f = pl.pallas_call(
    _tf_body,
    grid=(NRB,),
    in_specs=[
        pl.BlockSpec((RB, O), lambda r: (r, 0)),
        pl.BlockSpec((RB,), lambda r: (r,)),
        pl.BlockSpec((L + 1, O), lambda r: (0, 0)),
    ],
    out_specs=pl.BlockSpec((G, O), lambda r: (0, 0)),
    out_shape=jax.ShapeDtypeStruct((G, O), jnp.float32),
)


def kernel(x, edge_index, batch, Wc, bc, gamma, beta, Wf, bf):
    f32 = jnp.float32
    src = edge_index[0]
    dst = edge_index[1]
    x_p = jnp.pad(x, ((0, NP - N), (0, 0)))
    batch_p = jnp.pad(batch, (0, NP - N), constant_values=G - 1)
    src2d = jnp.pad(src, (0, EP - E)).reshape(EP // EC, EC)
    dst2d = jnp.pad(dst, (0, EP - E), constant_values=NP - 1).reshape(EP // EC, EC)
    zeros_hd = jnp.zeros((RPT, HD), f32)
    ones_np = jnp.ones((NP, HD), f32)

    sc_scatter = _sc_kernels()
    deg_lo, _unused = sc_scatter(ones_np, ones_np, src2d, dst2d, zeros_hd)
    dinv = _t_dinv(deg_lo)
    hn_lo, hn_hi, z = _t_a(x_p, Wc[0], Wf[0], dinv)
    for i in range(L):
        acc_lo, acc_hi = sc_scatter(hn_lo, hn_hi, src2d, dst2d, zeros_hd)
        y, sums = _t_b(acc_lo, acc_hi, hn_lo, hn_hi, dinv, bc[i])
        if i < L - 1:
            hn_lo, hn_hi, z = _t_cz(y, sums, gamma[i], beta[i], Wc[i + 1],
                                    Wf[i + 1], z, dinv)
        else:
            z = _t_cf(y, sums, gamma[i], beta[i], Wf[i + 1], z)
    return _t_f(z, batch_p, bf)


# speedup vs baseline: 6.0179x; 1.3064x over previous
import functools

import jax
import jax.numpy as jnp
from jax import lax
from jax.experimental import pallas as pl
from jax.experimental.pallas import tpu as pltpu
from jax.experimental.pallas import tpu_sc as plsc

N = 10000
NP = 10240
E = 160000
EP = 163840
D = 256
HD = 128
O = 10
G = 128
L = 4
EPS = 1e-5
RB = 512
NRB = NP // RB
RPT = NP // 16
EC = 128
_PREC = lax.Precision.HIGHEST

def _sc_scatter_body(hn_lo, hn_hi, src2d, dst2d, zeros_hbm, out_lo, out_hi,
                     src_v, dst_v, buf0, buf1, acc, sem):
    cid = lax.axis_index("c")
    sid = lax.axis_index("s")
    sl = pl.ds(sid * RPT, RPT)
    pltpu.sync_copy(zeros_hbm, acc.at[sl])
    plsc.subcore_barrier()

    def run(hn_ref):
        @pl.loop(0, 2)
        def _(p):
            base = sid * 80 + p * 40
            pltpu.sync_copy(src2d.at[pl.ds(base, 40)], src_v)
            pltpu.sync_copy(dst2d.at[pl.ds(base, 40)], dst_v)
            pltpu.async_copy(hn_ref.at[src_v.at[0]], buf0, sem)

            @pl.loop(0, 20)
            def _(i):
                j0 = 2 * i
                pltpu.make_async_copy(hn_ref.at[src_v.at[j0]], buf0, sem).wait()
                pltpu.async_copy(hn_ref.at[src_v.at[j0 + 1]], buf1, sem)
                pltpu.sync_copy(buf0, acc.at[dst_v.at[j0]], add=True)
                pltpu.make_async_copy(hn_ref.at[src_v.at[j0 + 1]], buf1, sem).wait()

                @pl.when(i < 19)
                def _():
                    pltpu.async_copy(hn_ref.at[src_v.at[j0 + 2]], buf0, sem)

                pltpu.sync_copy(buf1, acc.at[dst_v.at[j0 + 1]], add=True)

    @pl.when(cid == 0)
    def _():
        run(hn_lo)

    @pl.when(cid == 1)
    def _():
        run(hn_hi)

    plsc.subcore_barrier()

    @pl.when(cid == 0)
    def _():
        pltpu.sync_copy(acc.at[sl], out_lo.at[sl])

    @pl.when(cid == 1)
    def _():
        pltpu.sync_copy(acc.at[sl], out_hi.at[sl])


def _sc_deg_body(dst2d, ones_hbm, zeros_hbm, out0, out1, dst_v, ones_v, acc):
    cid = lax.axis_index("c")
    sid = lax.axis_index("s")
    sl = pl.ds(sid * RPT, RPT)
    pltpu.sync_copy(zeros_hbm, acc.at[sl])
    pltpu.sync_copy(dst2d.at[pl.ds((cid * 16 + sid) * 40, 40)], dst_v)
    pltpu.sync_copy(ones_hbm, ones_v)
    plsc.subcore_barrier()

    @pl.loop(0, 40)
    def _(j):
        pltpu.sync_copy(ones_v, acc.at[dst_v.at[j]], add=True)

    plsc.subcore_barrier()

    @pl.when(cid == 0)
    def _():
        pltpu.sync_copy(acc.at[sl], out0.at[sl])

    @pl.when(cid == 1)
    def _():
        pltpu.sync_copy(acc.at[sl], out1.at[sl])


@functools.cache
def _sc_kernels():
    mesh = plsc.VectorSubcoreMesh(core_axis_name="c", subcore_axis_name="s")
    sc_scatter = pl.kernel(
        _sc_scatter_body,
        out_type=(jax.ShapeDtypeStruct((NP, HD), jnp.float32),
                  jax.ShapeDtypeStruct((NP, HD), jnp.float32)),
        mesh=mesh,
        scratch_types=[
            pltpu.VMEM((40, EC), jnp.int32),
            pltpu.VMEM((40, EC), jnp.int32),
            pltpu.VMEM((EC, HD), jnp.float32),
            pltpu.VMEM((EC, HD), jnp.float32),
            pltpu.VMEM_SHARED((NP, HD), jnp.float32),
            pltpu.SemaphoreType.DMA,
        ],
    )
    sc_deg = pl.kernel(
        _sc_deg_body,
        out_type=(jax.ShapeDtypeStruct((NP, HD), jnp.float32),
                  jax.ShapeDtypeStruct((NP, HD), jnp.float32)),
        mesh=mesh,
        scratch_types=[
            pltpu.VMEM((40, EC), jnp.int32),
            pltpu.VMEM((EC, HD), jnp.float32),
            pltpu.VMEM_SHARED((NP, HD), jnp.float32),
        ],
    )
    return sc_deg, sc_scatter


def _dinv_body(deg0_ref, deg1_ref, dinv_ref):
    deg = (jnp.sum(deg0_ref[...], axis=1) + jnp.sum(deg1_ref[...], axis=1)) * (1.0 / HD)
    dinv_ref[...] = lax.rsqrt(1.0 + deg)


_t_dinv = pl.pallas_call(
    _dinv_body,
    out_shape=jax.ShapeDtypeStruct((NP,), jnp.float32),
)


def _ta_body(x_ref, w_ref, wf_ref, dinv_ref, hnlo_ref, hnhi_ref, z_ref):
    xb = x_ref[...]
    hn = jnp.dot(xb, w_ref[...], precision=_PREC) * dinv_ref[...][:, None]
    hnlo_ref[...] = hn[:, :HD]
    hnhi_ref[...] = hn[:, HD:]
    z_ref[...] = jnp.dot(xb, wf_ref[...], precision=_PREC)


_t_a = pl.pallas_call(
    _ta_body,
    grid=(NRB,),
    in_specs=[
        pl.BlockSpec((RB, D), lambda r: (r, 0)),
        pl.BlockSpec((D, D), lambda r: (0, 0)),
        pl.BlockSpec((D, O), lambda r: (0, 0)),
        pl.BlockSpec((RB,), lambda r: (r,)),
    ],
    out_specs=[
        pl.BlockSpec((RB, HD), lambda r: (r, 0)),
        pl.BlockSpec((RB, HD), lambda r: (r, 0)),
        pl.BlockSpec((RB, O), lambda r: (r, 0)),
    ],
    out_shape=[
        jax.ShapeDtypeStruct((NP, HD), jnp.float32),
        jax.ShapeDtypeStruct((NP, HD), jnp.float32),
        jax.ShapeDtypeStruct((NP, O), jnp.float32),
    ],
)


def _tb_body(acclo_ref, acchi_ref, hnlo_ref, hnhi_ref, dinv_ref, b_ref,
             y_ref, sums_ref):
    r = pl.program_id(0)
    a = jnp.concatenate(
        [acclo_ref[...] + hnlo_ref[...], acchi_ref[...] + hnhi_ref[...]], axis=1)
    y = a * dinv_ref[...][:, None] + b_ref[...][None, :]
    rowid = lax.broadcasted_iota(jnp.int32, (RB, 1), 0) + r * RB
    y = jnp.where(rowid < N, y, 0.0)
    y_ref[...] = y

    @pl.when(r == 0)
    def _():
        sums_ref[...] = jnp.zeros_like(sums_ref)

    sums_ref[...] += jnp.stack([jnp.sum(y, axis=0), jnp.sum(y * y, axis=0)])


_t_b = pl.pallas_call(
    _tb_body,
    grid=(NRB,),
    in_specs=[
        pl.BlockSpec((RB, HD), lambda r: (r, 0)),
        pl.BlockSpec((RB, HD), lambda r: (r, 0)),
        pl.BlockSpec((RB, HD), lambda r: (r, 0)),
        pl.BlockSpec((RB, HD), lambda r: (r, 0)),
        pl.BlockSpec((RB,), lambda r: (r,)),
        pl.BlockSpec((D,), lambda r: (0,)),
    ],
    out_specs=[
        pl.BlockSpec((RB, D), lambda r: (r, 0)),
        pl.BlockSpec((2, D), lambda r: (0, 0)),
    ],
    out_shape=[
        jax.ShapeDtypeStruct((NP, D), jnp.float32),
        jax.ShapeDtypeStruct((2, D), jnp.float32),
    ],
)


def _norm_relu(y, sums, g, be, r):
    m = sums[0] * (1.0 / N)
    var = sums[1] * (1.0 / N) - m * m
    rstd = lax.rsqrt(var + EPS)
    scale = g * rstd
    shift = be - m * scale
    h = jnp.maximum(y * scale[None, :] + shift[None, :], 0.0)
    rowid = lax.broadcasted_iota(jnp.int32, (RB, 1), 0) + r * RB
    return jnp.where(rowid < N, h, 0.0)


def _tcz_body(y_ref, sums_ref, g_ref, be_ref, w_ref, wf_ref, zin_ref, dinv_ref,
              hnlo_ref, hnhi_ref, z_ref):
    r = pl.program_id(0)
    h = _norm_relu(y_ref[...], sums_ref[...], g_ref[...], be_ref[...], r)
    hn = jnp.dot(h, w_ref[...], precision=_PREC) * dinv_ref[...][:, None]
    hnlo_ref[...] = hn[:, :HD]
    hnhi_ref[...] = hn[:, HD:]
    z_ref[...] = zin_ref[...] + jnp.dot(h, wf_ref[...], precision=_PREC)


_t_cz = pl.pallas_call(
    _tcz_body,
    grid=(NRB,),
    in_specs=[
        pl.BlockSpec((RB, D), lambda r: (r, 0)),
        pl.BlockSpec((2, D), lambda r: (0, 0)),
        pl.BlockSpec((D,), lambda r: (0,)),
        pl.BlockSpec((D,), lambda r: (0,)),
        pl.BlockSpec((D, D), lambda r: (0, 0)),
        pl.BlockSpec((D, O), lambda r: (0, 0)),
        pl.BlockSpec((RB, O), lambda r: (r, 0)),
        pl.BlockSpec((RB,), lambda r: (r,)),
    ],
    out_specs=[
        pl.BlockSpec((RB, HD), lambda r: (r, 0)),
        pl.BlockSpec((RB, HD), lambda r: (r, 0)),
        pl.BlockSpec((RB, O), lambda r: (r, 0)),
    ],
    out_shape=[
        jax.ShapeDtypeStruct((NP, HD), jnp.float32),
        jax.ShapeDtypeStruct((NP, HD), jnp.float32),
        jax.ShapeDtypeStruct((NP, O), jnp.float32),
    ],
)


def _tcf_body(y_ref, sums_ref, g_ref, be_ref, wf_ref, zin_ref, z_ref):
    r = pl.program_id(0)
    h = _norm_relu(y_ref[...], sums_ref[...], g_ref[...], be_ref[...], r)
    z_ref[...] = zin_ref[...] + jnp.dot(h, wf_ref[...], precision=_PREC)


_t_cf = pl.pallas_call(
    _tcf_body,
    grid=(NRB,),
    in_specs=[
        pl.BlockSpec((RB, D), lambda r: (r, 0)),
        pl.BlockSpec((2, D), lambda r: (0, 0)),
        pl.BlockSpec((D,), lambda r: (0,)),
        pl.BlockSpec((D,), lambda r: (0,)),
        pl.BlockSpec((D, O), lambda r: (0, 0)),
        pl.BlockSpec((RB, O), lambda r: (r, 0)),
    ],
    out_specs=pl.BlockSpec((RB, O), lambda r: (r, 0)),
    out_shape=jax.ShapeDtypeStruct((NP, O), jnp.float32),
)


def _tf_body(z_ref, batch_ref, bf_ref, out_ref):
    r = pl.program_id(0)
    gids = lax.broadcasted_iota(jnp.int32, (G, RB), 0)
    oh = (gids == batch_ref[...][None, :]).astype(jnp.float32)
    part = jnp.dot(oh, z_ref[...], precision=_PREC)

    @pl.when(r == 0)
    def _():
        out_ref[...] = jnp.zeros_like(out_ref)

    out_ref[...] += part

    @pl.when(r == NRB - 1)
    def _():
        t = out_ref[...] + jnp.sum(bf_ref[...], axis=0)[None, :]
        mx = jnp.max(t, axis=1, keepdims=True)
        lse = jnp.log(jnp.sum(jnp.exp(t - mx), axis=1, keepdims=True))
        out_ref[...] = t - mx - lse


_t_f = pl.pallas_call(
    _tf_body,
    grid=(NRB,),
    in_specs=[
        pl.BlockSpec((RB, O), lambda r: (r, 0)),
        pl.BlockSpec((RB,), lambda r: (r,)),
        pl.BlockSpec((L + 1, O), lambda r: (0, 0)),
    ],
    out_specs=pl.BlockSpec((G, O), lambda r: (0, 0)),
    out_shape=jax.ShapeDtypeStruct((G, O), jnp.float32),
)


def kernel(x, edge_index, batch, Wc, bc, gamma, beta, Wf, bf):
    f32 = jnp.float32
    src = edge_index[0]
    dst = edge_index[1]
    x_p = jnp.pad(x, ((0, NP - N), (0, 0)))
    batch_p = jnp.pad(batch, (0, NP - N), constant_values=G - 1)
    src2d = jnp.pad(src, (0, EP - E)).reshape(EP // EC, EC)
    dst2d = jnp.pad(dst, (0, EP - E), constant_values=NP - 1).reshape(EP // EC, EC)
    zeros_hd = jnp.zeros((RPT, HD), f32)
    ones_ec = jnp.ones((EC, HD), f32)

    sc_deg, sc_scatter = _sc_kernels()
    deg0, deg1 = sc_deg(dst2d, ones_ec, zeros_hd)
    dinv = _t_dinv(deg0, deg1)
    hn_lo, hn_hi, z = _t_a(x_p, Wc[0], Wf[0], dinv)
    for i in range(L):
        acc_lo, acc_hi = sc_scatter(hn_lo, hn_hi, src2d, dst2d, zeros_hd)
        y, sums = _t_b(acc_lo, acc_hi, hn_lo, hn_hi, dinv, bc[i])
        if i < L - 1:
            hn_lo, hn_hi, z = _t_cz(y, sums, gamma[i], beta[i], Wc[i + 1],
                                    Wf[i + 1], z, dinv)
        else:
            z = _t_cf(y, sums, gamma[i], beta[i], Wf[i + 1], z)
    return _t_f(z, batch_p, bf)


# speedup vs baseline: 6.7809x; 1.1268x over previous
import functools

import jax
import jax.numpy as jnp
from jax import lax
from jax.experimental import pallas as pl
from jax.experimental.pallas import tpu as pltpu
from jax.experimental.pallas import tpu_sc as plsc

N = 10000
NP = 10240
E = 160000
EP = 163840
D = 256
HD = 128
O = 10
G = 128
L = 4
EPS = 1e-5
RB = 512
NRB = NP // RB
RPT = NP // 16
EC = 128
ECS = 64
ECH = 80
_PREC = lax.Precision.HIGHEST

def _sc_scatter_body(hn_lo, hn_hi, src2d, dst2d, zeros_hbm, out_lo, out_hi,
                     src_v, dst_v, buf0, buf1, buf2,
                     acc, sem0, sem1, sem2):
    cid = lax.axis_index("c")
    sid = lax.axis_index("s")
    sl = pl.ds(sid * RPT, RPT)
    pltpu.sync_copy(zeros_hbm, acc.at[sl])
    plsc.subcore_barrier()
    bufs = (buf0, buf1, buf2)
    sems = (sem0, sem1, sem2)

    def run(hn_ref):
        @pl.loop(0, 2)
        def _(p):
            base = sid * (2 * ECH) + p * ECH
            pltpu.sync_copy(src2d.at[pl.ds(base, ECH)], src_v)
            pltpu.sync_copy(dst2d.at[pl.ds(base, ECH)], dst_v)
            for k in range(2):
                pltpu.async_copy(hn_ref.at[src_v.at[k]], bufs[k], sems[k])

            @pl.loop(0, (ECH - 2) // 3)
            def _(i):
                c0 = 3 * i

                def step(k):
                    c = c0 + k
                    pltpu.make_async_copy(
                        hn_ref.at[src_v.at[c]], bufs[k], sems[k]).wait()
                    kn = (k + 2) % 3
                    pltpu.async_copy(
                        hn_ref.at[src_v.at[c + 2]], bufs[kn], sems[kn])
                    pltpu.sync_copy(bufs[k], acc.at[dst_v.at[c]], add=True)

                for k in range(3):
                    step(k)

            for c, k in ((ECH - 2, 0), (ECH - 1, 1)):
                pltpu.make_async_copy(
                    hn_ref.at[src_v.at[c]], bufs[k], sems[k]).wait()
                pltpu.sync_copy(bufs[k], acc.at[dst_v.at[c]], add=True)

    @pl.when(cid == 0)
    def _():
        run(hn_lo)

    @pl.when(cid == 1)
    def _():
        run(hn_hi)

    plsc.subcore_barrier()

    @pl.when(cid == 0)
    def _():
        pltpu.sync_copy(acc.at[sl], out_lo.at[sl])

    @pl.when(cid == 1)
    def _():
        pltpu.sync_copy(acc.at[sl], out_hi.at[sl])


def _sc_deg_body(dst2d, ones_hbm, zeros_hbm, out0, out1, dst_v, ones_v, acc):
    cid = lax.axis_index("c")
    sid = lax.axis_index("s")
    sl = pl.ds(sid * RPT, RPT)
    pltpu.sync_copy(zeros_hbm, acc.at[sl])
    pltpu.sync_copy(dst2d.at[pl.ds((cid * 16 + sid) * 40, 40)], dst_v)
    pltpu.sync_copy(ones_hbm, ones_v)
    plsc.subcore_barrier()

    @pl.loop(0, 40)
    def _(j):
        pltpu.sync_copy(ones_v, acc.at[dst_v.at[j]], add=True)

    plsc.subcore_barrier()

    @pl.when(cid == 0)
    def _():
        pltpu.sync_copy(acc.at[sl], out0.at[sl])

    @pl.when(cid == 1)
    def _():
        pltpu.sync_copy(acc.at[sl], out1.at[sl])


@functools.cache
def _sc_kernels():
    mesh = plsc.VectorSubcoreMesh(core_axis_name="c", subcore_axis_name="s")
    sc_scatter = pl.kernel(
        _sc_scatter_body,
        out_type=(jax.ShapeDtypeStruct((NP, HD), jnp.float32),
                  jax.ShapeDtypeStruct((NP, HD), jnp.float32)),
        mesh=mesh,
        scratch_types=[
            pltpu.VMEM((ECH, ECS), jnp.int32),
            pltpu.VMEM((ECH, ECS), jnp.int32),
            pltpu.VMEM((ECS, HD), jnp.float32),
            pltpu.VMEM((ECS, HD), jnp.float32),
            pltpu.VMEM((ECS, HD), jnp.float32),
            pltpu.VMEM_SHARED((NP, HD), jnp.float32),
            pltpu.SemaphoreType.DMA,
            pltpu.SemaphoreType.DMA,
            pltpu.SemaphoreType.DMA,
        ],
    )
    sc_deg = pl.kernel(
        _sc_deg_body,
        out_type=(jax.ShapeDtypeStruct((NP, HD), jnp.float32),
                  jax.ShapeDtypeStruct((NP, HD), jnp.float32)),
        mesh=mesh,
        scratch_types=[
            pltpu.VMEM((40, EC), jnp.int32),
            pltpu.VMEM((EC, HD), jnp.float32),
            pltpu.VMEM_SHARED((NP, HD), jnp.float32),
        ],
    )
    return sc_deg, sc_scatter


def _dinv_body(deg0_ref, deg1_ref, dinv_ref):
    deg = (jnp.sum(deg0_ref[...], axis=1) + jnp.sum(deg1_ref[...], axis=1)) * (1.0 / HD)
    dinv_ref[...] = lax.rsqrt(1.0 + deg)


_t_dinv = pl.pallas_call(
    _dinv_body,
    out_shape=jax.ShapeDtypeStruct((NP,), jnp.float32),
)


def _ta_body(x_ref, w_ref, wf_ref, dinv_ref, hnlo_ref, hnhi_ref, z_ref):
    xb = x_ref[...]
    hn = jnp.dot(xb, w_ref[...], precision=_PREC) * dinv_ref[...][:, None]
    hnlo_ref[...] = hn[:, :HD]
    hnhi_ref[...] = hn[:, HD:]
    z_ref[...] = jnp.dot(xb, wf_ref[...], precision=_PREC)


_t_a = pl.pallas_call(
    _ta_body,
    grid=(NRB,),
    in_specs=[
        pl.BlockSpec((RB, D), lambda r: (r, 0)),
        pl.BlockSpec((D, D), lambda r: (0, 0)),
        pl.BlockSpec((D, O), lambda r: (0, 0)),
        pl.BlockSpec((RB,), lambda r: (r,)),
    ],
    out_specs=[
        pl.BlockSpec((RB, HD), lambda r: (r, 0)),
        pl.BlockSpec((RB, HD), lambda r: (r, 0)),
        pl.BlockSpec((RB, O), lambda r: (r, 0)),
    ],
    out_shape=[
        jax.ShapeDtypeStruct((NP, HD), jnp.float32),
        jax.ShapeDtypeStruct((NP, HD), jnp.float32),
        jax.ShapeDtypeStruct((NP, O), jnp.float32),
    ],
)


def _tb_body(acclo_ref, acchi_ref, hnlo_ref, hnhi_ref, dinv_ref, b_ref,
             y_ref, sums_ref):
    r = pl.program_id(0)
    a = jnp.concatenate(
        [acclo_ref[...] + hnlo_ref[...], acchi_ref[...] + hnhi_ref[...]], axis=1)
    y = a * dinv_ref[...][:, None] + b_ref[...][None, :]
    rowid = lax.broadcasted_iota(jnp.int32, (RB, 1), 0) + r * RB
    y = jnp.where(rowid < N, y, 0.0)
    y_ref[...] = y

    @pl.when(r == 0)
    def _():
        sums_ref[...] = jnp.zeros_like(sums_ref)

    sums_ref[...] += jnp.stack([jnp.sum(y, axis=0), jnp.sum(y * y, axis=0)])


_t_b = pl.pallas_call(
    _tb_body,
    grid=(NRB,),
    in_specs=[
        pl.BlockSpec((RB, HD), lambda r: (r, 0)),
        pl.BlockSpec((RB, HD), lambda r: (r, 0)),
        pl.BlockSpec((RB, HD), lambda r: (r, 0)),
        pl.BlockSpec((RB, HD), lambda r: (r, 0)),
        pl.BlockSpec((RB,), lambda r: (r,)),
        pl.BlockSpec((D,), lambda r: (0,)),
    ],
    out_specs=[
        pl.BlockSpec((RB, D), lambda r: (r, 0)),
        pl.BlockSpec((2, D), lambda r: (0, 0)),
    ],
    out_shape=[
        jax.ShapeDtypeStruct((NP, D), jnp.float32),
        jax.ShapeDtypeStruct((2, D), jnp.float32),
    ],
)


def _norm_relu(y, sums, g, be, r):
    m = sums[0] * (1.0 / N)
    var = sums[1] * (1.0 / N) - m * m
    rstd = lax.rsqrt(var + EPS)
    scale = g * rstd
    shift = be - m * scale
    h = jnp.maximum(y * scale[None, :] + shift[None, :], 0.0)
    rowid = lax.broadcasted_iota(jnp.int32, (RB, 1), 0) + r * RB
    return jnp.where(rowid < N, h, 0.0)


def _tcz_body(y_ref, sums_ref, g_ref, be_ref, w_ref, wf_ref, zin_ref, dinv_ref,
              hnlo_ref, hnhi_ref, z_ref):
    r = pl.program_id(0)
    h = _norm_relu(y_ref[...], sums_ref[...], g_ref[...], be_ref[...], r)
    hn = jnp.dot(h, w_ref[...], precision=_PREC) * dinv_ref[...][:, None]
    hnlo_ref[...] = hn[:, :HD]
    hnhi_ref[...] = hn[:, HD:]
    z_ref[...] = zin_ref[...] + jnp.dot(h, wf_ref[...], precision=_PREC)


_t_cz = pl.pallas_call(
    _tcz_body,
    grid=(NRB,),
    in_specs=[
        pl.BlockSpec((RB, D), lambda r: (r, 0)),
        pl.BlockSpec((2, D), lambda r: (0, 0)),
        pl.BlockSpec((D,), lambda r: (0,)),
        pl.BlockSpec((D,), lambda r: (0,)),
        pl.BlockSpec((D, D), lambda r: (0, 0)),
        pl.BlockSpec((D, O), lambda r: (0, 0)),
        pl.BlockSpec((RB, O), lambda r: (r, 0)),
        pl.BlockSpec((RB,), lambda r: (r,)),
    ],
    out_specs=[
        pl.BlockSpec((RB, HD), lambda r: (r, 0)),
        pl.BlockSpec((RB, HD), lambda r: (r, 0)),
        pl.BlockSpec((RB, O), lambda r: (r, 0)),
    ],
    out_shape=[
        jax.ShapeDtypeStruct((NP, HD), jnp.float32),
        jax.ShapeDtypeStruct((NP, HD), jnp.float32),
        jax.ShapeDtypeStruct((NP, O), jnp.float32),
    ],
)


def _tcf_body(y_ref, sums_ref, g_ref, be_ref, wf_ref, zin_ref, z_ref):
    r = pl.program_id(0)
    h = _norm_relu(y_ref[...], sums_ref[...], g_ref[...], be_ref[...], r)
    z_ref[...] = zin_ref[...] + jnp.dot(h, wf_ref[...], precision=_PREC)


_t_cf = pl.pallas_call(
    _tcf_body,
    grid=(NRB,),
    in_specs=[
        pl.BlockSpec((RB, D), lambda r: (r, 0)),
        pl.BlockSpec((2, D), lambda r: (0, 0)),
        pl.BlockSpec((D,), lambda r: (0,)),
        pl.BlockSpec((D,), lambda r: (0,)),
        pl.BlockSpec((D, O), lambda r: (0, 0)),
        pl.BlockSpec((RB, O), lambda r: (r, 0)),
    ],
    out_specs=pl.BlockSpec((RB, O), lambda r: (r, 0)),
    out_shape=jax.ShapeDtypeStruct((NP, O), jnp.float32),
)


def _tf_body(z_ref, batch_ref, bf_ref, out_ref):
    r = pl.program_id(0)
    gids = lax.broadcasted_iota(jnp.int32, (G, RB), 0)
    oh = (gids == batch_ref[...][None, :]).astype(jnp.float32)
    part = jnp.dot(oh, z_ref[...], precision=_PREC)

    @pl.when(r == 0)
    def _():
        out_ref[...] = jnp.zeros_like(out_ref)

    out_ref[...] += part

    @pl.when(r == NRB - 1)
    def _():
        t = out_ref[...] + jnp.sum(bf_ref[...], axis=0)[None, :]
        mx = jnp.max(t, axis=1, keepdims=True)
        lse = jnp.log(jnp.sum(jnp.exp(t - mx), axis=1, keepdims=True))
        out_ref[...] = t - mx - lse


_t_f = pl.pallas_call(
    _tf_body,
    grid=(NRB,),
    in_specs=[
        pl.BlockSpec((RB, O), lambda r: (r, 0)),
        pl.BlockSpec((RB,), lambda r: (r,)),
        pl.BlockSpec((L + 1, O), lambda r: (0, 0)),
    ],
    out_specs=pl.BlockSpec((G, O), lambda r: (0, 0)),
    out_shape=jax.ShapeDtypeStruct((G, O), jnp.float32),
)


def kernel(x, edge_index, batch, Wc, bc, gamma, beta, Wf, bf):
    f32 = jnp.float32
    src = edge_index[0]
    dst = edge_index[1]
    x_p = jnp.pad(x, ((0, NP - N), (0, 0)))
    batch_p = jnp.pad(batch, (0, NP - N), constant_values=G - 1)
    src_p = jnp.pad(src, (0, EP - E))
    dst_p = jnp.pad(dst, (0, EP - E), constant_values=NP - 1)
    dst2d = dst_p.reshape(EP // EC, EC)
    src64 = src_p.reshape(EP // ECS, ECS)
    dst64 = dst_p.reshape(EP // ECS, ECS)
    zeros_hd = jnp.zeros((RPT, HD), f32)
    ones_ec = jnp.ones((EC, HD), f32)

    sc_deg, sc_scatter = _sc_kernels()
    deg0, deg1 = sc_deg(dst2d, ones_ec, zeros_hd)
    dinv = _t_dinv(deg0, deg1)
    hn_lo, hn_hi, z = _t_a(x_p, Wc[0], Wf[0], dinv)
    for i in range(L):
        acc_lo, acc_hi = sc_scatter(hn_lo, hn_hi, src64, dst64, zeros_hd)
        y, sums = _t_b(acc_lo, acc_hi, hn_lo, hn_hi, dinv, bc[i])
        if i < L - 1:
            hn_lo, hn_hi, z = _t_cz(y, sums, gamma[i], beta[i], Wc[i + 1],
                                    Wf[i + 1], z, dinv)
        else:
            z = _t_cf(y, sums, gamma[i], beta[i], Wf[i + 1], z)
    return _t_f(z, batch_p, bf)
